# baseline (speedup 1.0000x reference)
import jax
import jax.numpy as jnp
from jax.experimental import pallas as pl

NUM_CODES = 1024
DIM = 64


def _vq_block(z_ref, sz_ref, sw_ref, w_ref, zq_ref, idx_ref):
    zb = z_ref[0]
    w = w_ref[...]
    sz = sz_ref[0]
    sw = sw_ref[...]
    m = jax.lax.dot_general(
        zb.astype(jnp.bfloat16), w.astype(jnp.bfloat16), (((1,), (1,)), ((), ())),
        preferred_element_type=jnp.float32,
    )
    d = (sz + sw) - 2.0 * m
    dmin = jnp.min(d, axis=1, keepdims=True)
    iota = jax.lax.broadcasted_iota(jnp.int32, (zb.shape[0], NUM_CODES), 1)
    idx = jnp.min(jnp.where(d == dmin, iota, NUM_CODES), axis=1).astype(jnp.int32)
    onehot = (iota == idx[:, None]).astype(jnp.float32)
    zq = jax.lax.dot_general(
        onehot, w, (((1,), (0,)), ((), ())),
        preferred_element_type=jnp.float32,
        precision=jax.lax.Precision.HIGHEST,
    )
    zq_ref[...] = zq[None]
    idx_ref[...] = idx.reshape(1, 1, NUM_CODES)


def kernel(z, W):
    B, HW, D = z.shape
    sz = jnp.sum(z ** 2, axis=-1, keepdims=True)
    sw = jnp.sum(W ** 2, axis=1).reshape(1, NUM_CODES)
    zq, idx3 = pl.pallas_call(
        _vq_block,
        grid=(B,),
        in_specs=[
            pl.BlockSpec((1, HW, D), lambda b: (b, 0, 0)),
            pl.BlockSpec((1, HW, 1), lambda b: (b, 0, 0)),
            pl.BlockSpec((1, NUM_CODES), lambda b: (0, 0)),
            pl.BlockSpec((NUM_CODES, D), lambda b: (0, 0)),
        ],
        out_specs=[
            pl.BlockSpec((1, HW, D), lambda b: (b, 0, 0)),
            pl.BlockSpec((1, 1, HW), lambda b: (b, 0, 0)),
        ],
        out_shape=[
            jax.ShapeDtypeStruct((B, HW, D), jnp.float32),
            jax.ShapeDtypeStruct((B, 1, HW), jnp.int32),
        ],
    )(z, sz, sw, W)
    return zq, idx3.reshape(B, HW)


# speedup vs baseline: 1.2606x; 1.2606x over previous
import jax
import jax.numpy as jnp
from jax.experimental import pallas as pl

NUM_CODES = 1024
DIM = 64


def _vq_block(z_ref, sz_ref, sw_ref, w_ref, zq_ref, idx_ref):
    zb = z_ref[0]
    w = w_ref[...]
    sz = sz_ref[0]
    sw = sw_ref[...]
    m = jax.lax.dot_general(
        zb.astype(jnp.bfloat16), w.astype(jnp.bfloat16), (((1,), (1,)), ((), ())),
        preferred_element_type=jnp.float32,
    )
    d = (sz + sw) - 2.0 * m
    dmin = jnp.min(d, axis=1, keepdims=True)
    iota = jax.lax.broadcasted_iota(jnp.int32, (zb.shape[0], NUM_CODES), 1)
    idx = jnp.min(jnp.where(d == dmin, iota, NUM_CODES), axis=1).astype(jnp.int32)
    onehot = (iota == idx[:, None]).astype(jnp.bfloat16)
    w_hi = w.astype(jnp.bfloat16)
    r1 = w - w_hi.astype(jnp.float32)
    w_mid = r1.astype(jnp.bfloat16)
    w_lo = (r1 - w_mid.astype(jnp.float32)).astype(jnp.bfloat16)

    def sel(wp):
        return jax.lax.dot_general(
            onehot, wp, (((1,), (0,)), ((), ())),
            preferred_element_type=jnp.float32,
        )

    zq = (sel(w_hi) + sel(w_mid)) + sel(w_lo)
    zq_ref[...] = zq[None]
    idx_ref[...] = idx.reshape(1, 1, NUM_CODES)


def kernel(z, W):
    B, HW, D = z.shape
    sz = jnp.sum(z ** 2, axis=-1, keepdims=True)
    sw = jnp.sum(W ** 2, axis=1).reshape(1, NUM_CODES)
    zq, idx3 = pl.pallas_call(
        _vq_block,
        grid=(B,),
        in_specs=[
            pl.BlockSpec((1, HW, D), lambda b: (b, 0, 0)),
            pl.BlockSpec((1, HW, 1), lambda b: (b, 0, 0)),
            pl.BlockSpec((1, NUM_CODES), lambda b: (0, 0)),
            pl.BlockSpec((NUM_CODES, D), lambda b: (0, 0)),
        ],
        out_specs=[
            pl.BlockSpec((1, HW, D), lambda b: (b, 0, 0)),
            pl.BlockSpec((1, 1, HW), lambda b: (b, 0, 0)),
        ],
        out_shape=[
            jax.ShapeDtypeStruct((B, HW, D), jnp.float32),
            jax.ShapeDtypeStruct((B, 1, HW), jnp.int32),
        ],
    )(z, sz, sw, W)
    return zq, idx3.reshape(B, HW)
